# baseline (speedup 1.0000x reference)
import jax
import jax.numpy as jnp
from jax import lax
from jax.experimental import pallas as pl
from jax.experimental.pallas import tpu as pltpu
from jax.experimental.pallas import tpu_sc as plsc

N = 10000
F = 128
E = 320000
CHUNK = 125
NSUB = 16
NW = 2 * NSUB
EPT = E // NW
NCHUNK = EPT // CHUNK
IBLK = 8
NIB = NCHUNK // IBLK
ZROWS = 40
NZCH = N // ZROWS
PREC = lax.Precision.DEFAULT


def _sc_agg_body(x, src2, dst2, zeros_hbm, out,
                 sb0, sb1, db0, db1, rows0, rows1, acc,
                 sem0, sem1, isem0, isem1):
    cid = lax.axis_index("c")
    sid = lax.axis_index("s")
    wid = cid * NSUB + sid
    base = wid * NCHUNK

    pltpu.sync_copy(zeros_hbm, rows0.at[pl.ds(0, ZROWS)])
    for k in range((NZCH + NSUB - 1) // NSUB):
        c = sid + k * NSUB

        @pl.when(c < NZCH)
        def _zero(c=c):
            pltpu.sync_copy(rows0.at[pl.ds(0, ZROWS)],
                            acc.at[pl.ds(c * ZROWS, ZROWS)])

    plsc.subcore_barrier()

    sbs = (sb0, sb1)
    dbs = (db0, db1)
    isems = (isem0, isem1)
    bufs = (rows0, rows1)
    sems = (sem0, sem1)

    def idx_dma(k, kb, which):
        arr = (src2, dst2)[which]
        buf = (sbs, dbs)[which][kb]
        return pltpu.make_async_copy(
            arr.at[pl.ds(base + k * IBLK, IBLK)], buf, isems[kb])

    for w in range(2):
        idx_dma(0, 0, w).start()
        idx_dma(1, 1, w).start()

    def outer(t, carry):
        for kb in range(2):
            k = 2 * t + kb
            idx_dma(k, kb, 0).wait()
            idx_dma(k, kb, 1).wait()
            sb, db = sbs[kb], dbs[kb]
            pltpu.async_copy(x.at[sb.at[0]], rows0, sem0)
            pltpu.async_copy(x.at[sb.at[1]], rows1, sem1)
            for cc in range(IBLK):
                b = cc % 2
                pltpu.make_async_copy(x.at[sb.at[cc]], bufs[b], sems[b]).wait()
                pltpu.sync_copy(bufs[b], acc.at[db.at[cc]], add=True)
                if cc + 2 < IBLK:
                    pltpu.async_copy(x.at[sb.at[cc + 2]], bufs[b], sems[b])

            @pl.when(k + 2 < NIB)
            def _prefetch(k=k, kb=kb):
                idx_dma(k + 2, kb, 0).start()
                idx_dma(k + 2, kb, 1).start()
        return carry

    lax.fori_loop(0, NIB // 2, outer, None)
    plsc.subcore_barrier()
    for k in range((NZCH + NSUB - 1) // NSUB):
        c = sid + k * NSUB

        @pl.when(c < NZCH)
        def _wb(c=c):
            pltpu.sync_copy(acc.at[pl.ds(c * ZROWS, ZROWS)],
                            rows0.at[pl.ds(0, ZROWS)])
            pltpu.sync_copy(rows0.at[pl.ds(0, ZROWS)],
                            out.at[cid, pl.ds(c * ZROWS, ZROWS)])


def _sc_agg2_body(xm, xd, sm2, dm2, sd2, dd2, zeros_hbm, agg_m, agg_d,
                  sb0, sb1, db0, db1, rows0, rows1, acc,
                  sem0, sem1, isem0, isem1):
    cid = lax.axis_index("c")
    sid = lax.axis_index("s")

    sbs = (sb0, sb1)
    dbs = (db0, db1)
    isems = (isem0, isem1)
    bufs = (rows0, rows1)
    sems = (sem0, sem1)
    nchunk = 2 * NCHUNK
    nib = nchunk // IBLK

    def run(x, src2, dst2, out):
        base = sid * nchunk

        pltpu.sync_copy(zeros_hbm, rows0.at[pl.ds(0, ZROWS)])
        for k in range((NZCH + NSUB - 1) // NSUB):
            c = sid + k * NSUB

            @pl.when(c < NZCH)
            def _zero(c=c):
                pltpu.sync_copy(rows0.at[pl.ds(0, ZROWS)],
                                acc.at[pl.ds(c * ZROWS, ZROWS)])

        plsc.subcore_barrier()

        def idx_dma(k, kb, which):
            arr = (src2, dst2)[which]
            buf = (sbs, dbs)[which][kb]
            return pltpu.make_async_copy(
                arr.at[pl.ds(base + k * IBLK, IBLK)], buf, isems[kb])

        for w in range(2):
            idx_dma(0, 0, w).start()
            idx_dma(1, 1, w).start()

        def outer(t, carry):
            for kb in range(2):
                k = 2 * t + kb
                idx_dma(k, kb, 0).wait()
                idx_dma(k, kb, 1).wait()
                sb, db = sbs[kb], dbs[kb]
                pltpu.async_copy(x.at[sb.at[0]], rows0, sem0)
                pltpu.async_copy(x.at[sb.at[1]], rows1, sem1)
                for cc in range(IBLK):
                    b = cc % 2
                    pltpu.make_async_copy(x.at[sb.at[cc]], bufs[b],
                                          sems[b]).wait()
                    pltpu.sync_copy(bufs[b], acc.at[db.at[cc]], add=True)
                    if cc + 2 < IBLK:
                        pltpu.async_copy(x.at[sb.at[cc + 2]], bufs[b], sems[b])

                @pl.when(k + 2 < nib)
                def _prefetch(k=k, kb=kb):
                    idx_dma(k + 2, kb, 0).start()
                    idx_dma(k + 2, kb, 1).start()
            return carry

        lax.fori_loop(0, nib // 2, outer, None)
        plsc.subcore_barrier()
        for k in range((NZCH + NSUB - 1) // NSUB):
            c = sid + k * NSUB

            @pl.when(c < NZCH)
            def _wb(c=c):
                pltpu.sync_copy(acc.at[pl.ds(c * ZROWS, ZROWS)],
                                rows0.at[pl.ds(0, ZROWS)])
                pltpu.sync_copy(rows0.at[pl.ds(0, ZROWS)],
                                out.at[pl.ds(c * ZROWS, ZROWS)])

    @pl.when(cid == 0)
    def _m():
        run(xm, sm2, dm2, agg_m)

    @pl.when(cid == 1)
    def _d():
        run(xd, sd2, dd2, agg_d)


def _sc_agg_dual(xm, xd, sm2, dm2, sd2, dd2, zeros_hbm):
    return pl.kernel(
        _sc_agg2_body,
        out_type=(
            jax.ShapeDtypeStruct((N, F), jnp.float32),
            jax.ShapeDtypeStruct((N, F), jnp.float32),
        ),
        mesh=plsc.VectorSubcoreMesh(core_axis_name="c", subcore_axis_name="s"),
        scratch_types=[
            pltpu.VMEM((IBLK, CHUNK), jnp.int32),
            pltpu.VMEM((IBLK, CHUNK), jnp.int32),
            pltpu.VMEM((IBLK, CHUNK), jnp.int32),
            pltpu.VMEM((IBLK, CHUNK), jnp.int32),
            pltpu.VMEM((CHUNK, F), jnp.float32),
            pltpu.VMEM((CHUNK, F), jnp.float32),
            pltpu.VMEM_SHARED((N, F), jnp.float32),
            pltpu.SemaphoreType.DMA,
            pltpu.SemaphoreType.DMA,
            pltpu.SemaphoreType.DMA,
            pltpu.SemaphoreType.DMA,
        ],
    )(xm, xd, sm2, dm2, sd2, dd2, zeros_hbm)


def _sc_agg(x, src2, dst2, zeros_hbm):
    return pl.kernel(
        _sc_agg_body,
        out_type=jax.ShapeDtypeStruct((2, N, F), jnp.float32),
        mesh=plsc.VectorSubcoreMesh(core_axis_name="c", subcore_axis_name="s"),
        scratch_types=[
            pltpu.VMEM((IBLK, CHUNK), jnp.int32),
            pltpu.VMEM((IBLK, CHUNK), jnp.int32),
            pltpu.VMEM((IBLK, CHUNK), jnp.int32),
            pltpu.VMEM((IBLK, CHUNK), jnp.int32),
            pltpu.VMEM((CHUNK, F), jnp.float32),
            pltpu.VMEM((CHUNK, F), jnp.float32),
            pltpu.VMEM_SHARED((N, F), jnp.float32),
            pltpu.SemaphoreType.DMA,
            pltpu.SemaphoreType.DMA,
            pltpu.SemaphoreType.DMA,
            pltpu.SemaphoreType.DMA,
        ],
    )(x, src2, dst2, zeros_hbm)


def _fold_body(w1a, b1a, w1b, b1b, w2a, b2a, w2b, b2b,
               w1e, b1e, w2e, b2e):
    w1e[...] = jnp.dot(w1a[...], w1b[...], preferred_element_type=jnp.float32,
                       precision=PREC)
    b1e[...] = jnp.dot(b1a[...], w1b[...], preferred_element_type=jnp.float32,
                       precision=PREC) + b1b[...]
    w2e[...] = jnp.dot(w2a[...], w2b[...], preferred_element_type=jnp.float32,
                       precision=PREC)
    b2e[...] = jnp.dot(b2a[...], w2b[...], preferred_element_type=jnp.float32,
                       precision=PREC) + b2b[...]


def _fold(w1a, b1a, w1b, b1b, w2a, b2a, w2b, b2b):
    f1, f2 = w1a.shape[0], w1b.shape[1]
    return pl.pallas_call(
        _fold_body,
        out_shape=(
            jax.ShapeDtypeStruct((f1, f2), jnp.float32),
            jax.ShapeDtypeStruct((1, f2), jnp.float32),
            jax.ShapeDtypeStruct((f2, f1), jnp.float32),
            jax.ShapeDtypeStruct((1, f1), jnp.float32),
        ),
    )(w1a, b1a.reshape(1, -1), w1b, b1b.reshape(1, -1),
      w2a, b2a.reshape(1, -1), w2b, b2b.reshape(1, -1))


BLK = 1000


def _gin_body(eps, x, agg, w1e, b1e, w2e, z):
    a = agg[0]
    for p in range(1, agg.shape[0]):
        a = a + agg[p]
    u = (1.0 + eps[0, 0]) * x[...] + a
    h = jnp.dot(u, w1e[...], preferred_element_type=jnp.float32, precision=PREC)
    h = jnp.maximum(h + b1e[...], 0.0)
    z[...] = jnp.dot(h, w2e[...], preferred_element_type=jnp.float32,
                     precision=PREC)


def _gin_block(eps, x, agg, w1e, b1e, w2e):
    f1, f2 = w1e.shape
    np_ = agg.shape[0]
    return pl.pallas_call(
        _gin_body,
        grid=(N // BLK,),
        in_specs=[
            pl.BlockSpec((1, 1), lambda i: (0, 0)),
            pl.BlockSpec((BLK, f1), lambda i: (i, 0)),
            pl.BlockSpec((np_, BLK, f1), lambda i: (0, i, 0)),
            pl.BlockSpec((f1, f2), lambda i: (0, 0)),
            pl.BlockSpec((1, f2), lambda i: (0, 0)),
            pl.BlockSpec((f2, f1), lambda i: (0, 0)),
        ],
        out_specs=pl.BlockSpec((BLK, f1), lambda i: (i, 0)),
        out_shape=jax.ShapeDtypeStruct((N, f1), jnp.float32),
    )(eps.reshape(1, 1), x, agg, w1e, b1e, w2e)


def _post_body(eps, z, agg, b2e, wl1, bl1, wl2, bl2, wl3, bl3, out):
    h = jnp.maximum((1.0 + eps[0, 0]) * z[...] + (agg[0] + agg[1]) + b2e[...],
                    0.0)
    h = jnp.maximum(jnp.dot(h, wl1[...], preferred_element_type=jnp.float32,
                            precision=PREC) + bl1[...], 0.0)
    h = jnp.maximum(jnp.dot(h, wl2[...], preferred_element_type=jnp.float32,
                            precision=PREC) + bl2[...], 0.0)
    out[...] = jnp.maximum(jnp.dot(h, wl3[...], preferred_element_type=jnp.float32,
                                   precision=PREC) + bl3[...], 0.0)


def _post_block(eps, z, agg, b2e, wl1, bl1, wl2, bl2, wl3, bl3):
    k = wl3.shape[1]
    return pl.pallas_call(
        _post_body,
        grid=(N // BLK,),
        in_specs=[
            pl.BlockSpec((1, 1), lambda i: (0, 0)),
            pl.BlockSpec((BLK, F), lambda i: (i, 0)),
            pl.BlockSpec((2, BLK, F), lambda i: (0, i, 0)),
            pl.BlockSpec((1, F), lambda i: (0, 0)),
            pl.BlockSpec(wl1.shape, lambda i: (0, 0)),
            pl.BlockSpec((1, wl1.shape[1]), lambda i: (0, 0)),
            pl.BlockSpec(wl2.shape, lambda i: (0, 0)),
            pl.BlockSpec((1, wl2.shape[1]), lambda i: (0, 0)),
            pl.BlockSpec(wl3.shape, lambda i: (0, 0)),
            pl.BlockSpec((1, k), lambda i: (0, 0)),
        ],
        out_specs=pl.BlockSpec((BLK, k), lambda i: (i, 0)),
        out_shape=jax.ShapeDtypeStruct((N, k), jnp.float32),
    )(eps.reshape(1, 1), z, agg, b2e,
      wl1, bl1.reshape(1, -1), wl2, bl2.reshape(1, -1), wl3, bl3.reshape(1, -1))


def _outer_body(fx, fy, o):
    o[...] = lax.dot_general(fx[...], fy[...], (((1,), (1,)), ((), ())),
                             preferred_element_type=jnp.float32,
                             precision=PREC)


OBLK = 400


def _outer(fx, fy):
    k = fx.shape[1]
    return pl.pallas_call(
        _outer_body,
        grid=(N // OBLK,),
        in_specs=[
            pl.BlockSpec((OBLK, k), lambda i: (i, 0)),
            pl.BlockSpec((N, k), lambda i: (0, 0)),
        ],
        out_specs=pl.BlockSpec((OBLK, N), lambda i: (i, 0)),
        out_shape=jax.ShapeDtypeStruct((N, N), jnp.float32),
    )(fx, fy)


def kernel(x_m, x_d, mm_edge_index, dd_edge_index,
           W_x1a, b_x1a, W_x1b, b_x1b, W_x2a, b_x2a, W_x2b, b_x2b,
           W_y1a, b_y1a, W_y1b, b_y1b, W_y2a, b_y2a, W_y2b, b_y2b,
           W_lx1, b_lx1, W_lx2, b_lx2, W_lx3, b_lx3,
           W_ly1, b_ly1, W_ly2, b_ly2, W_ly3, b_ly3,
           eps_x1, eps_x2, eps_y1, eps_y2):
    src_m = mm_edge_index[0].reshape(E // CHUNK, CHUNK)
    dst_m = mm_edge_index[1].reshape(E // CHUNK, CHUNK)
    src_d = dd_edge_index[0].reshape(E // CHUNK, CHUNK)
    dst_d = dd_edge_index[1].reshape(E // CHUNK, CHUNK)
    zeros_hbm = jnp.zeros((ZROWS, F), jnp.float32)

    w1em, b1em, w2em, b2em = _fold(W_x1a, b_x1a, W_x1b, b_x1b,
                                   W_x2a, b_x2a, W_x2b, b_x2b)
    w1ed, b1ed, w2ed, b2ed = _fold(W_y1a, b_y1a, W_y1b, b_y1b,
                                   W_y2a, b_y2a, W_y2b, b_y2b)

    agg_m, agg_d = _sc_agg_dual(x_m, x_d, src_m, dst_m, src_d, dst_d,
                                zeros_hbm)
    z_m = _gin_block(eps_x1, x_m, agg_m[None], w1em, b1em, w2em)
    z_d = _gin_block(eps_y1, x_d, agg_d[None], w1ed, b1ed, w2ed)
    agg_zm = _sc_agg(z_m, src_m, dst_m, zeros_hbm)
    agg_zd = _sc_agg(z_d, src_d, dst_d, zeros_hbm)
    fx = _post_block(eps_x2, z_m, agg_zm, b2em,
                     W_lx1, b_lx1, W_lx2, b_lx2, W_lx3, b_lx3)
    fy = _post_block(eps_y2, z_d, agg_zd, b2ed,
                     W_ly1, b_ly1, W_ly2, b_ly2, W_ly3, b_ly3)
    return _outer(fx, fy)


# speedup vs baseline: 1.0113x; 1.0113x over previous
import jax
import jax.numpy as jnp
from jax import lax
from jax.experimental import pallas as pl
from jax.experimental.pallas import tpu as pltpu
from jax.experimental.pallas import tpu_sc as plsc

N = 10000
F = 128
E = 320000
CHUNK = 125
NSUB = 16
NW = 2 * NSUB
EPT = E // NW
NCHUNK = EPT // CHUNK
IBLK = 8
NIB = NCHUNK // IBLK
ZROWS = 40
NZCH = N // ZROWS
PREC = lax.Precision.DEFAULT


def _sc_agg_body(x, src2, dst2, zeros_hbm, out,
                 sb0, sb1, db0, db1, rows0, rows1, acc,
                 sem0, sem1, isem0, isem1):
    cid = lax.axis_index("c")
    sid = lax.axis_index("s")
    wid = cid * NSUB + sid
    base = wid * NCHUNK

    pltpu.sync_copy(zeros_hbm, rows0.at[pl.ds(0, ZROWS)])
    for k in range((NZCH + NSUB - 1) // NSUB):
        c = sid + k * NSUB

        @pl.when(c < NZCH)
        def _zero(c=c):
            pltpu.sync_copy(rows0.at[pl.ds(0, ZROWS)],
                            acc.at[pl.ds(c * ZROWS, ZROWS)])

    plsc.subcore_barrier()

    sbs = (sb0, sb1)
    dbs = (db0, db1)
    isems = (isem0, isem1)
    bufs = (rows0, rows1)
    sems = (sem0, sem1)

    def idx_dma(k, kb, which):
        arr = (src2, dst2)[which]
        buf = (sbs, dbs)[which][kb]
        return pltpu.make_async_copy(
            arr.at[pl.ds(base + k * IBLK, IBLK)], buf, isems[kb])

    for w in range(2):
        idx_dma(0, 0, w).start()
        idx_dma(1, 1, w).start()

    def outer(t, carry):
        for kb in range(2):
            k = 2 * t + kb
            idx_dma(k, kb, 0).wait()
            idx_dma(k, kb, 1).wait()
            sb, db = sbs[kb], dbs[kb]
            pltpu.async_copy(x.at[sb.at[0]], rows0, sem0)
            pltpu.async_copy(x.at[sb.at[1]], rows1, sem1)
            for cc in range(IBLK):
                b = cc % 2
                pltpu.make_async_copy(x.at[sb.at[cc]], bufs[b], sems[b]).wait()
                pltpu.sync_copy(bufs[b], acc.at[db.at[cc]], add=True)
                if cc + 2 < IBLK:
                    pltpu.async_copy(x.at[sb.at[cc + 2]], bufs[b], sems[b])

            @pl.when(k + 2 < NIB)
            def _prefetch(k=k, kb=kb):
                idx_dma(k + 2, kb, 0).start()
                idx_dma(k + 2, kb, 1).start()
        return carry

    lax.fori_loop(0, NIB // 2, outer, None)
    plsc.subcore_barrier()
    for k in range((NZCH + NSUB - 1) // NSUB):
        c = sid + k * NSUB

        @pl.when(c < NZCH)
        def _wb(c=c):
            pltpu.sync_copy(acc.at[pl.ds(c * ZROWS, ZROWS)],
                            rows0.at[pl.ds(0, ZROWS)])
            pltpu.sync_copy(rows0.at[pl.ds(0, ZROWS)],
                            out.at[cid, pl.ds(c * ZROWS, ZROWS)])


def _sc_agg(x, src2, dst2, zeros_hbm):
    return pl.kernel(
        _sc_agg_body,
        out_type=jax.ShapeDtypeStruct((2, N, F), jnp.float32),
        mesh=plsc.VectorSubcoreMesh(core_axis_name="c", subcore_axis_name="s"),
        scratch_types=[
            pltpu.VMEM((IBLK, CHUNK), jnp.int32),
            pltpu.VMEM((IBLK, CHUNK), jnp.int32),
            pltpu.VMEM((IBLK, CHUNK), jnp.int32),
            pltpu.VMEM((IBLK, CHUNK), jnp.int32),
            pltpu.VMEM((CHUNK, F), jnp.float32),
            pltpu.VMEM((CHUNK, F), jnp.float32),
            pltpu.VMEM_SHARED((N, F), jnp.float32),
            pltpu.SemaphoreType.DMA,
            pltpu.SemaphoreType.DMA,
            pltpu.SemaphoreType.DMA,
            pltpu.SemaphoreType.DMA,
        ],
    )(x, src2, dst2, zeros_hbm)


def _fold_body(w1a, b1a, w1b, b1b, w2a, b2a, w2b, b2b,
               w1e, b1e, w2e, b2e):
    w1e[...] = jnp.dot(w1a[...], w1b[...], preferred_element_type=jnp.float32,
                       precision=PREC)
    b1e[...] = jnp.dot(b1a[...], w1b[...], preferred_element_type=jnp.float32,
                       precision=PREC) + b1b[...]
    w2e[...] = jnp.dot(w2a[...], w2b[...], preferred_element_type=jnp.float32,
                       precision=PREC)
    b2e[...] = jnp.dot(b2a[...], w2b[...], preferred_element_type=jnp.float32,
                       precision=PREC) + b2b[...]


def _fold(w1a, b1a, w1b, b1b, w2a, b2a, w2b, b2b):
    f1, f2 = w1a.shape[0], w1b.shape[1]
    return pl.pallas_call(
        _fold_body,
        out_shape=(
            jax.ShapeDtypeStruct((f1, f2), jnp.float32),
            jax.ShapeDtypeStruct((1, f2), jnp.float32),
            jax.ShapeDtypeStruct((f2, f1), jnp.float32),
            jax.ShapeDtypeStruct((1, f1), jnp.float32),
        ),
    )(w1a, b1a.reshape(1, -1), w1b, b1b.reshape(1, -1),
      w2a, b2a.reshape(1, -1), w2b, b2b.reshape(1, -1))


BLK = 1000


def _gin_body(eps, x, agg, w1e, b1e, w2e, z):
    u = (1.0 + eps[0, 0]) * x[...] + (agg[0] + agg[1])
    h = jnp.dot(u, w1e[...], preferred_element_type=jnp.float32, precision=PREC)
    h = jnp.maximum(h + b1e[...], 0.0)
    z[...] = jnp.dot(h, w2e[...], preferred_element_type=jnp.float32,
                     precision=PREC)


def _gin_block(eps, x, agg, w1e, b1e, w2e):
    f1, f2 = w1e.shape
    return pl.pallas_call(
        _gin_body,
        grid=(N // BLK,),
        in_specs=[
            pl.BlockSpec((1, 1), lambda i: (0, 0)),
            pl.BlockSpec((BLK, f1), lambda i: (i, 0)),
            pl.BlockSpec((2, BLK, f1), lambda i: (0, i, 0)),
            pl.BlockSpec((f1, f2), lambda i: (0, 0)),
            pl.BlockSpec((1, f2), lambda i: (0, 0)),
            pl.BlockSpec((f2, f1), lambda i: (0, 0)),
        ],
        out_specs=pl.BlockSpec((BLK, f1), lambda i: (i, 0)),
        out_shape=jax.ShapeDtypeStruct((N, f1), jnp.float32),
    )(eps.reshape(1, 1), x, agg, w1e, b1e, w2e)


def _post_body(eps, z, agg, b2e, wl1, bl1, wl2, bl2, wl3, bl3, out):
    h = jnp.maximum((1.0 + eps[0, 0]) * z[...] + (agg[0] + agg[1]) + b2e[...],
                    0.0)
    h = jnp.maximum(jnp.dot(h, wl1[...], preferred_element_type=jnp.float32,
                            precision=PREC) + bl1[...], 0.0)
    h = jnp.maximum(jnp.dot(h, wl2[...], preferred_element_type=jnp.float32,
                            precision=PREC) + bl2[...], 0.0)
    out[...] = jnp.maximum(jnp.dot(h, wl3[...], preferred_element_type=jnp.float32,
                                   precision=PREC) + bl3[...], 0.0)


def _post_block(eps, z, agg, b2e, wl1, bl1, wl2, bl2, wl3, bl3):
    k = wl3.shape[1]
    return pl.pallas_call(
        _post_body,
        grid=(N // BLK,),
        in_specs=[
            pl.BlockSpec((1, 1), lambda i: (0, 0)),
            pl.BlockSpec((BLK, F), lambda i: (i, 0)),
            pl.BlockSpec((2, BLK, F), lambda i: (0, i, 0)),
            pl.BlockSpec((1, F), lambda i: (0, 0)),
            pl.BlockSpec(wl1.shape, lambda i: (0, 0)),
            pl.BlockSpec((1, wl1.shape[1]), lambda i: (0, 0)),
            pl.BlockSpec(wl2.shape, lambda i: (0, 0)),
            pl.BlockSpec((1, wl2.shape[1]), lambda i: (0, 0)),
            pl.BlockSpec(wl3.shape, lambda i: (0, 0)),
            pl.BlockSpec((1, k), lambda i: (0, 0)),
        ],
        out_specs=pl.BlockSpec((BLK, k), lambda i: (i, 0)),
        out_shape=jax.ShapeDtypeStruct((N, k), jnp.float32),
    )(eps.reshape(1, 1), z, agg, b2e,
      wl1, bl1.reshape(1, -1), wl2, bl2.reshape(1, -1), wl3, bl3.reshape(1, -1))


def _outer_body(fx, fy, o):
    o[...] = lax.dot_general(fx[...], fy[...], (((1,), (1,)), ((), ())),
                             preferred_element_type=jnp.float32,
                             precision=PREC)


OBLK = 400


def _outer(fx, fy):
    k = fx.shape[1]
    return pl.pallas_call(
        _outer_body,
        grid=(N // OBLK,),
        in_specs=[
            pl.BlockSpec((OBLK, k), lambda i: (i, 0)),
            pl.BlockSpec((N, k), lambda i: (0, 0)),
        ],
        out_specs=pl.BlockSpec((OBLK, N), lambda i: (i, 0)),
        out_shape=jax.ShapeDtypeStruct((N, N), jnp.float32),
    )(fx, fy)


def kernel(x_m, x_d, mm_edge_index, dd_edge_index,
           W_x1a, b_x1a, W_x1b, b_x1b, W_x2a, b_x2a, W_x2b, b_x2b,
           W_y1a, b_y1a, W_y1b, b_y1b, W_y2a, b_y2a, W_y2b, b_y2b,
           W_lx1, b_lx1, W_lx2, b_lx2, W_lx3, b_lx3,
           W_ly1, b_ly1, W_ly2, b_ly2, W_ly3, b_ly3,
           eps_x1, eps_x2, eps_y1, eps_y2):
    src_m = mm_edge_index[0].reshape(E // CHUNK, CHUNK)
    dst_m = mm_edge_index[1].reshape(E // CHUNK, CHUNK)
    src_d = dd_edge_index[0].reshape(E // CHUNK, CHUNK)
    dst_d = dd_edge_index[1].reshape(E // CHUNK, CHUNK)
    zeros_hbm = jnp.zeros((ZROWS, F), jnp.float32)

    w1em, b1em, w2em, b2em = _fold(W_x1a, b_x1a, W_x1b, b_x1b,
                                   W_x2a, b_x2a, W_x2b, b_x2b)
    w1ed, b1ed, w2ed, b2ed = _fold(W_y1a, b_y1a, W_y1b, b_y1b,
                                   W_y2a, b_y2a, W_y2b, b_y2b)

    agg_m = _sc_agg(x_m, src_m, dst_m, zeros_hbm)
    agg_d = _sc_agg(x_d, src_d, dst_d, zeros_hbm)
    z_m = _gin_block(eps_x1, x_m, agg_m, w1em, b1em, w2em)
    z_d = _gin_block(eps_y1, x_d, agg_d, w1ed, b1ed, w2ed)
    agg_zm = _sc_agg(z_m, src_m, dst_m, zeros_hbm)
    agg_zd = _sc_agg(z_d, src_d, dst_d, zeros_hbm)
    fx = _post_block(eps_x2, z_m, agg_zm, b2em,
                     W_lx1, b_lx1, W_lx2, b_lx2, W_lx3, b_lx3)
    fy = _post_block(eps_y2, z_d, agg_zd, b2ed,
                     W_ly1, b_ly1, W_ly2, b_ly2, W_ly3, b_ly3)
    return _outer(fx, fy)
